# baseline (speedup 1.0000x reference)
import functools
import math

import jax
import jax.numpy as jnp
from jax import lax
from jax.experimental import pallas as pl
from jax.experimental.pallas import tpu as pltpu
from jax.experimental.pallas import tpu_sc as plsc


def _pgen_body(T, S, ds_ref, sm_ref, wq_ref, wk_ref, bq_ref, bk_ref,
               w1_ref, w2_ref, bp_ref, p_ref):
    BT, D = ds_ref.shape
    BS = sm_ref.shape[0]
    ds = ds_ref[...]
    sm = sm_ref[...]
    q = jnp.dot(ds, wq_ref[...], preferred_element_type=jnp.float32) + bq_ref[...]
    k = jnp.dot(sm, wk_ref[...], preferred_element_type=jnp.float32) + bk_ref[...]
    scores = jax.lax.dot_general(
        q, k, (((1,), (1,)), ((), ())),
        preferred_element_type=jnp.float32) * (1.0 / math.sqrt(D))
    rb = jax.lax.broadcasted_iota(jnp.int32, (BT, BS), 0) // T
    cb = jax.lax.broadcasted_iota(jnp.int32, (BT, BS), 1) // S
    scores = jnp.where(rb == cb, scores, -1e30)
    m = jnp.max(scores, axis=1, keepdims=True)
    e = jnp.exp(scores - m)
    attn = e / jnp.sum(e, axis=1, keepdims=True)
    kv = jnp.sum(sm * w2_ref[...], axis=1, keepdims=True)
    ctx = jnp.dot(attn, kv, preferred_element_type=jnp.float32)
    dsw = jnp.sum(ds * w1_ref[...], axis=1, keepdims=True)
    logit = (dsw + ctx + bp_ref[0, 0] - 0.5) * 10.0
    p_ref[...] = jax.nn.sigmoid(logit)


def _make_sc_fill(BT, Vx):
    info = plsc.get_sparse_core_info()
    NC, NS, L = info.num_cores, info.num_subcores, info.num_lanes
    NW = NC * NS
    rows_per_w = BT // NW
    mesh = plsc.VectorSubcoreMesh(core_axis_name="c", subcore_axis_name="s")

    @functools.partial(
        pl.kernel, mesh=mesh,
        out_type=jax.ShapeDtypeStruct((BT, Vx), jnp.float32),
        scratch_types=[
            pltpu.VMEM((1, Vx), jnp.float32),
            pltpu.SemaphoreType.DMA,
        ],
    )
    def fill(out_hbm, zrow, sem):
        wid = lax.axis_index("s") * NC + lax.axis_index("c")
        base = wid * rows_per_w
        zero16 = jnp.zeros((L,), jnp.float32)

        def zstep(i, _):
            zrow[0, pl.ds(i * L, L)] = zero16
            return 0

        lax.fori_loop(0, Vx // L, zstep, 0)
        cps = [
            pltpu.make_async_copy(
                zrow, out_hbm.at[pl.ds(base + r, 1), :], sem)
            for r in range(rows_per_w)
        ]
        for c in cps:
            c.start()
        for c in cps:
            c.wait()

    return fill


def kernel(decoder_states, scene_memory, triplets, tokenizer, embedding_weight,
           device, W_q, b_q, W_k, b_k, W_pgen, b_pgen):
    Bx, Tx, Dx = decoder_states.shape
    Sx = scene_memory.shape[1]
    Vx = embedding_weight.shape[0]
    BT = Bx * Tx
    BS = Bx * Sx

    ds = decoder_states.reshape(BT, Dx)
    sm = scene_memory.reshape(BS, Dx)
    w1 = W_pgen[:Dx, :].T
    w2 = W_pgen[Dx:, :].T
    bq = b_q.reshape(1, Dx)
    bk = b_k.reshape(1, Dx)
    bp = b_pgen.reshape(1, 1)

    p = pl.pallas_call(
        functools.partial(_pgen_body, Tx, Sx),
        out_shape=jax.ShapeDtypeStruct((BT, 1), jnp.float32),
    )(ds, sm, W_q, W_k, bq, bk, w1, w2, bp)

    fill = _make_sc_fill(BT, Vx)()

    return (p.reshape(Bx, Tx, 1), fill.reshape(Bx, Tx, Vx))


# speedup vs baseline: 2.0998x; 2.0998x over previous
import functools
import math

import jax
import jax.numpy as jnp
from jax.experimental import pallas as pl
from jax.experimental.pallas import tpu as pltpu


def _body(T, S, RB, ds_hbm, sm_hbm, wq_hbm, wk_hbm, bq_ref, bk_ref,
          w1_ref, w2_ref, bp_ref, p_ref, out_hbm,
          zbuf, ds_v, sm_v, wq_v, wk_v, in_sem, out_sem):
    BT, D = ds_v.shape
    BS = sm_v.shape[0]
    nblk = out_hbm.shape[0] // RB

    cps = [
        pltpu.make_async_copy(ds_hbm, ds_v, in_sem),
        pltpu.make_async_copy(sm_hbm, sm_v, in_sem),
        pltpu.make_async_copy(wq_hbm, wq_v, in_sem),
        pltpu.make_async_copy(wk_hbm, wk_v, in_sem),
    ]
    for c in cps:
        c.start()

    zbuf[...] = jnp.zeros(zbuf.shape, zbuf.dtype)
    fills = [
        pltpu.make_async_copy(zbuf, out_hbm.at[pl.ds(i * RB, RB), :], out_sem)
        for i in range(nblk)
    ]
    for f in fills:
        f.start()

    for c in cps:
        c.wait()

    ds = ds_v[...]
    sm = sm_v[...]
    q = jnp.dot(ds, wq_v[...], preferred_element_type=jnp.float32) + bq_ref[...]
    k = jnp.dot(sm, wk_v[...], preferred_element_type=jnp.float32) + bk_ref[...]
    scores = jax.lax.dot_general(
        q, k, (((1,), (1,)), ((), ())),
        preferred_element_type=jnp.float32) * (1.0 / math.sqrt(D))
    rb = jax.lax.broadcasted_iota(jnp.int32, (BT, BS), 0) // T
    cb = jax.lax.broadcasted_iota(jnp.int32, (BT, BS), 1) // S
    scores = jnp.where(rb == cb, scores, -1e30)
    m = jnp.max(scores, axis=1, keepdims=True)
    e = jnp.exp(scores - m)
    attn = e / jnp.sum(e, axis=1, keepdims=True)
    kv = jnp.sum(sm * w2_ref[...], axis=1, keepdims=True)
    ctx = jnp.dot(attn, kv, preferred_element_type=jnp.float32)
    dsw = jnp.sum(ds * w1_ref[...], axis=1, keepdims=True)
    logit = (dsw + ctx + bp_ref[0, 0] - 0.5) * 10.0
    p_ref[...] = jax.nn.sigmoid(logit)

    for f in fills:
        f.wait()


def kernel(decoder_states, scene_memory, triplets, tokenizer, embedding_weight,
           device, W_q, b_q, W_k, b_k, W_pgen, b_pgen):
    Bx, Tx, Dx = decoder_states.shape
    Sx = scene_memory.shape[1]
    Vx = embedding_weight.shape[0]
    BT = Bx * Tx
    BS = Bx * Sx

    ds = decoder_states.reshape(BT, Dx)
    sm = scene_memory.reshape(BS, Dx)
    w1 = W_pgen[:Dx, :].T
    w2 = W_pgen[Dx:, :].T
    bq = b_q.reshape(1, Dx)
    bk = b_k.reshape(1, Dx)
    bp = b_pgen.reshape(1, 1)

    RB = 32
    anyspec = pl.BlockSpec(memory_space=pl.ANY)
    vmem = pl.BlockSpec(memory_space=pltpu.MemorySpace.VMEM)
    p, fill = pl.pallas_call(
        functools.partial(_body, Tx, Sx, RB),
        in_specs=[anyspec, anyspec, anyspec, anyspec,
                  vmem, vmem, vmem, vmem, vmem],
        out_specs=[vmem, anyspec],
        out_shape=[
            jax.ShapeDtypeStruct((BT, 1), jnp.float32),
            jax.ShapeDtypeStruct((BT, Vx), jnp.float32),
        ],
        scratch_shapes=[
            pltpu.VMEM((RB, Vx), jnp.float32),
            pltpu.VMEM((BT, Dx), jnp.float32),
            pltpu.VMEM((BS, Dx), jnp.float32),
            pltpu.VMEM((Dx, Dx), jnp.float32),
            pltpu.VMEM((Dx, Dx), jnp.float32),
            pltpu.SemaphoreType.DMA,
            pltpu.SemaphoreType.DMA,
        ],
    )(ds, sm, W_q, W_k, bq, bk, w1, w2, bp)

    return (p.reshape(Bx, Tx, 1), fill.reshape(Bx, Tx, Vx))


# speedup vs baseline: 2.1167x; 1.0081x over previous
import functools
import math

import jax
import jax.numpy as jnp
from jax.experimental import pallas as pl
from jax.experimental.pallas import tpu as pltpu


def _body(T, S, RB, ds_hbm, sm_hbm, wq_hbm, wk_hbm, bq_ref, bk_ref,
          w1_ref, w2_ref, bp_ref, p_ref, out_hbm,
          zbuf, ds_v, sm_v, wq_v, wk_v, in_sem, out_sem):
    BT, D = ds_v.shape
    BS = sm_v.shape[0]
    nblk = out_hbm.shape[0] // RB

    cps = [
        pltpu.make_async_copy(ds_hbm, ds_v, in_sem),
        pltpu.make_async_copy(sm_hbm, sm_v, in_sem),
        pltpu.make_async_copy(wq_hbm, wq_v, in_sem),
        pltpu.make_async_copy(wk_hbm, wk_v, in_sem),
    ]
    for c in cps:
        c.start()

    zbuf[...] = jnp.zeros(zbuf.shape, zbuf.dtype)
    fills = [
        pltpu.make_async_copy(zbuf, out_hbm.at[pl.ds(i * RB, RB), :], out_sem)
        for i in range(nblk)
    ]
    for f in fills:
        f.start()

    for c in cps:
        c.wait()

    ds = ds_v[...]
    sm = sm_v[...]
    q = jnp.dot(ds, wq_v[...], preferred_element_type=jnp.float32) + bq_ref[...]
    k = jnp.dot(sm, wk_v[...], preferred_element_type=jnp.float32) + bk_ref[...]
    scores = jax.lax.dot_general(
        q, k, (((1,), (1,)), ((), ())),
        preferred_element_type=jnp.float32) * (1.0 / math.sqrt(D))
    rb = jax.lax.broadcasted_iota(jnp.int32, (BT, BS), 0) // T
    cb = jax.lax.broadcasted_iota(jnp.int32, (BT, BS), 1) // S
    scores = jnp.where(rb == cb, scores, -1e30)
    m = jnp.max(scores, axis=1, keepdims=True)
    e = jnp.exp(scores - m)
    attn = e / jnp.sum(e, axis=1, keepdims=True)
    kv = jnp.sum(sm * w2_ref[...], axis=1, keepdims=True)
    ctx = jnp.dot(attn, kv, preferred_element_type=jnp.float32)
    dsw = jnp.sum(ds * w1_ref[...], axis=1, keepdims=True)
    logit = (dsw + ctx + bp_ref[0, 0] - 0.5) * 10.0
    p_ref[...] = jax.nn.sigmoid(logit)

    for f in fills:
        f.wait()


def kernel(decoder_states, scene_memory, triplets, tokenizer, embedding_weight,
           device, W_q, b_q, W_k, b_k, W_pgen, b_pgen):
    Bx, Tx, Dx = decoder_states.shape
    Sx = scene_memory.shape[1]
    Vx = embedding_weight.shape[0]
    BT = Bx * Tx
    BS = Bx * Sx

    ds = decoder_states.reshape(BT, Dx)
    sm = scene_memory.reshape(BS, Dx)
    w1 = W_pgen[:Dx, :].T
    w2 = W_pgen[Dx:, :].T
    bq = b_q.reshape(1, Dx)
    bk = b_k.reshape(1, Dx)
    bp = b_pgen.reshape(1, 1)

    RB = 64
    anyspec = pl.BlockSpec(memory_space=pl.ANY)
    vmem = pl.BlockSpec(memory_space=pltpu.MemorySpace.VMEM)
    p, fill = pl.pallas_call(
        functools.partial(_body, Tx, Sx, RB),
        in_specs=[anyspec, anyspec, anyspec, anyspec,
                  vmem, vmem, vmem, vmem, vmem],
        out_specs=[vmem, anyspec],
        out_shape=[
            jax.ShapeDtypeStruct((BT, 1), jnp.float32),
            jax.ShapeDtypeStruct((BT, Vx), jnp.float32),
        ],
        scratch_shapes=[
            pltpu.VMEM((RB, Vx), jnp.float32),
            pltpu.VMEM((BT, Dx), jnp.float32),
            pltpu.VMEM((BS, Dx), jnp.float32),
            pltpu.VMEM((Dx, Dx), jnp.float32),
            pltpu.VMEM((Dx, Dx), jnp.float32),
            pltpu.SemaphoreType.DMA,
            pltpu.SemaphoreType.DMA,
        ],
    )(ds, sm, W_q, W_k, bq, bk, w1, w2, bp)

    return (p.reshape(Bx, Tx, 1), fill.reshape(Bx, Tx, Vx))
